# Initial kernel scaffold; baseline (speedup 1.0000x reference)
import jax
import jax.numpy as jnp
from jax.experimental import pallas as pl


def kernel(x, edge_index, W_e1, b_e1, W_e2, b_e2, W_a1, b_a1, W_a2, b_a2, W_s1, b_s1):
    raise NotImplementedError("write your pallas kernel here")


# speedup vs baseline: 7.4398x; 7.4398x over previous
import functools

import jax
import jax.numpy as jnp
from jax import lax
from jax.experimental import pallas as pl
from jax.experimental.pallas import tpu as pltpu
from jax.experimental.pallas import tpu_sc as plsc

N = 10000
NFEAT = 128
NHID = 64
NEDGE = 320000

NC = 2
NS = 16
NW = NC * NS
EB = 128
NE2 = NEDGE + N
NB = -(-NE2 // (NW * EB))
NB += NB % 2
NE_PAD = NW * EB * NB
N_PAD = 10240
RPT = N_PAD // NS
BM = 256


def _sc_mesh():
    return plsc.VectorSubcoreMesh(core_axis_name="c", subcore_axis_name="s")


@functools.partial(
    pl.kernel,
    out_type=jax.ShapeDtypeStruct((NC * N_PAD, NFEAT), jnp.float32),
    mesh=_sc_mesh(),
    scratch_types=[
        pltpu.VMEM((NB, EB), jnp.int32),
        pltpu.VMEM((EB, NFEAT), jnp.float32),
        pltpu.VMEM_SHARED((N_PAD, NFEAT), jnp.float32),
    ],
)
def _deg_kernel(dst_hbm, ones_hbm, zeros_hbm, out_hbm, idx_d, ones_v, acc):
    c = lax.axis_index("c")
    s = lax.axis_index("s")
    wid = c * NS + s
    row0 = s * RPT
    pltpu.sync_copy(zeros_hbm.at[pl.ds(row0, RPT)], acc.at[pl.ds(row0, RPT)])
    pltpu.sync_copy(dst_hbm.at[wid], idx_d)
    pltpu.sync_copy(ones_hbm, ones_v)
    plsc.subcore_barrier()

    def body(i, carry):
        pltpu.sync_copy(ones_v, acc.at[idx_d.at[i]], add=True)
        return carry

    lax.fori_loop(0, NB, body, 0)
    plsc.subcore_barrier()
    pltpu.sync_copy(acc.at[pl.ds(row0, RPT)],
                    out_hbm.at[pl.ds(c * N_PAD + row0, RPT)])


def _make_prop(D):
    @functools.partial(
        pl.kernel,
        out_type=jax.ShapeDtypeStruct((NC * N_PAD, D), jnp.float32),
        mesh=_sc_mesh(),
        scratch_types=[
            pltpu.VMEM((NB, EB), jnp.int32),
            pltpu.VMEM((NB, EB), jnp.int32),
            pltpu.VMEM((EB, D), jnp.float32),
            pltpu.VMEM_SHARED((N_PAD, D), jnp.float32),
            pltpu.SemaphoreType.DMA,
        ],
    )
    def prop(g_hbm, src_hbm, dst_hbm, zeros_hbm, out_hbm,
             idx_s, idx_d, rows0, acc, sem0):
        c = lax.axis_index("c")
        s = lax.axis_index("s")
        wid = c * NS + s
        row0 = s * RPT
        pltpu.sync_copy(zeros_hbm.at[pl.ds(row0, RPT)],
                        acc.at[pl.ds(row0, RPT)])
        pltpu.sync_copy(src_hbm.at[wid], idx_s)
        pltpu.sync_copy(dst_hbm.at[wid], idx_d)
        plsc.subcore_barrier()

        def body(i, carry):
            pltpu.async_copy(g_hbm.at[idx_s.at[i]], rows0, sem0).wait()
            pltpu.sync_copy(rows0, acc.at[idx_d.at[i]], add=True)
            return carry

        lax.fori_loop(0, NB, body, 0)
        plsc.subcore_barrier()
        pltpu.sync_copy(acc.at[pl.ds(row0, RPT)],
                        out_hbm.at[pl.ds(c * N_PAD + row0, RPT)])

    return prop


_prop128 = _make_prop(NFEAT)


_GRID = N_PAD // BM


def _row_spec(w, off=0):
    return pl.BlockSpec((BM, w), lambda i, o=off: (i + o, 0))


def _full_spec(r, w):
    return pl.BlockSpec((r, w), lambda i: (0, 0))


def _stage_a_body(d0, d1, x, w, dinv_o, g_o):
    deg = d0[:, 0:1] + d1[:, 0:1]
    dinv = lax.rsqrt(jnp.maximum(deg, 1.0))
    h = jnp.dot(x[...], w[...], preferred_element_type=jnp.float32)
    dinv_o[...] = dinv
    g_o[...] = jnp.concatenate(
        [h * dinv, jnp.zeros((BM, NFEAT - NHID), jnp.float32)], axis=1)


def _stage_a(deg_parts, x_pad, W):
    return pl.pallas_call(
        _stage_a_body,
        grid=(_GRID,),
        in_specs=[_row_spec(NFEAT), _row_spec(NFEAT, _GRID), _row_spec(NFEAT),
                  _full_spec(NFEAT, NHID)],
        out_specs=[_row_spec(1), _row_spec(NFEAT)],
        out_shape=[jax.ShapeDtypeStruct((N_PAD, 1), jnp.float32),
                   jax.ShapeDtypeStruct((N_PAD, NFEAT), jnp.float32)],
    )(deg_parts, deg_parts, x_pad, W)


def _stage_b_body(p0, p1, dinv, b, w, g_o):
    t = jax.nn.relu(dinv[...] * (p0[...] + p1[...])[:, :NHID] + b[...])
    g = dinv[...] * jnp.dot(t, w[...], preferred_element_type=jnp.float32)
    g_o[...] = jnp.concatenate(
        [g, jnp.zeros((BM, NFEAT - NHID), jnp.float32)], axis=1)


def _stage_b(parts, dinv, b, W):
    return pl.pallas_call(
        _stage_b_body,
        grid=(_GRID,),
        in_specs=[_row_spec(NFEAT), _row_spec(NFEAT, _GRID), _row_spec(1),
                  _full_spec(1, NHID), _full_spec(NHID, NHID)],
        out_specs=_row_spec(NFEAT),
        out_shape=jax.ShapeDtypeStruct((N_PAD, NFEAT), jnp.float32),
    )(parts, parts, dinv, b, W)


def _stage_b2_body(p0, p1, dinv, b, wa, ws, g_o):
    t = jax.nn.relu(dinv[...] * (p0[...] + p1[...])[:, :NHID] + b[...])
    ga = jnp.dot(t, wa[...], preferred_element_type=jnp.float32)
    gs = jnp.dot(t, ws[...], preferred_element_type=jnp.float32)
    g_o[...] = dinv[...] * jnp.concatenate([ga, gs], axis=1)


def _stage_b2(parts, dinv, b, Wa, Ws):
    return pl.pallas_call(
        _stage_b2_body,
        grid=(_GRID,),
        in_specs=[_row_spec(NFEAT), _row_spec(NFEAT, _GRID), _row_spec(1),
                  _full_spec(1, NHID), _full_spec(NHID, NHID),
                  _full_spec(NHID, NHID)],
        out_specs=_row_spec(NFEAT),
        out_shape=jax.ShapeDtypeStruct((N_PAD, NFEAT), jnp.float32),
    )(parts, parts, dinv, b, Wa, Ws)


def _stage_c_body(p0, p1, dinv, ba, bs, wa2, g_o, s_o):
    m = dinv[...] * (p0[...] + p1[...])
    xa = jax.nn.relu(m[:, :NHID] + ba[...])
    sv = jax.nn.relu(m[:, NHID:] + bs[...])
    g_o[...] = dinv[...] * jnp.dot(xa, wa2[...],
                                   preferred_element_type=jnp.float32)
    s_o[...] = sv


def _stage_c(parts, dinv, ba, bs, Wa2):
    return pl.pallas_call(
        _stage_c_body,
        grid=(_GRID,),
        in_specs=[_row_spec(NFEAT), _row_spec(NFEAT, _GRID), _row_spec(1),
                  _full_spec(1, NHID), _full_spec(1, NHID),
                  _full_spec(NHID, NFEAT)],
        out_specs=[_row_spec(NFEAT), _row_spec(NHID)],
        out_shape=[jax.ShapeDtypeStruct((N_PAD, NFEAT), jnp.float32),
                   jax.ShapeDtypeStruct((N_PAD, NHID), jnp.float32)],
    )(parts, parts, dinv, ba, bs, Wa2)


def _stage_d_body(p0, p1, dinv, b, xh_o):
    xh_o[...] = jax.nn.relu(dinv[...] * (p0[...] + p1[...]) + b[...])


def _stage_d(parts, dinv, b):
    return pl.pallas_call(
        _stage_d_body,
        grid=(_GRID,),
        in_specs=[_row_spec(NFEAT), _row_spec(NFEAT, _GRID), _row_spec(1),
                  _full_spec(1, NFEAT)],
        out_specs=_row_spec(NFEAT),
        out_shape=jax.ShapeDtypeStruct((N_PAD, NFEAT), jnp.float32),
    )(parts, parts, dinv, b)


def _ahat_body(a, bt, o):
    o[...] = jnp.dot(a[...], bt[...], preferred_element_type=jnp.float32)


def _ahat(s2, st):
    grid = -(-N // BM)
    return pl.pallas_call(
        _ahat_body,
        grid=(grid,),
        in_specs=[pl.BlockSpec((BM, NHID), lambda i: (i, 0)),
                  pl.BlockSpec((NHID, N), lambda i: (0, 0))],
        out_specs=pl.BlockSpec((BM, N), lambda i: (i, 0)),
        out_shape=jax.ShapeDtypeStruct((N, N), jnp.float32),
    )(s2, st)


def kernel(x, edge_index, W_e1, b_e1, W_e2, b_e2, W_a1, b_a1, W_a2, b_a2,
           W_s1, b_s1):
    sl = jnp.arange(N, dtype=edge_index.dtype)
    pad = jnp.full((NE_PAD - NE2,), N, edge_index.dtype)
    src = jnp.concatenate([edge_index[0], sl, pad]).reshape(NW, NB, EB)
    dst = jnp.concatenate([edge_index[1], sl, pad]).reshape(NW, NB, EB)

    x_pad = jnp.zeros((N_PAD, NFEAT), jnp.float32).at[:N].set(x)
    z128 = jnp.zeros((N_PAD, NFEAT), jnp.float32)
    ones128 = jnp.ones((EB, NFEAT), jnp.float32)

    deg_parts = _deg_kernel(dst, ones128, z128)
    dinv, g1 = _stage_a(deg_parts, x_pad, W_e1)

    p1 = _prop128(g1, src, dst, z128)
    g2 = _stage_b(p1, dinv, b_e1.reshape(1, -1), W_e2)

    p2 = _prop128(g2, src, dst, z128)
    g_as = _stage_b2(p2, dinv, b_e2.reshape(1, -1), W_a1, W_s1)

    p_as = _prop128(g_as, src, dst, z128)
    g_a2, s_pad = _stage_c(p_as, dinv, b_a1.reshape(1, -1),
                           b_s1.reshape(1, -1), W_a2)

    p_a2 = _prop128(g_a2, src, dst, z128)
    x_hat = _stage_d(p_a2, dinv, b_a2.reshape(1, -1))[:N]

    s2 = s_pad[:N]
    A_hat = _ahat(s2, s2.T)
    return (A_hat, x_hat)
